# Initial kernel scaffold; baseline (speedup 1.0000x reference)
import jax
import jax.numpy as jnp
from jax.experimental import pallas as pl


def kernel(X, table, W1, b1, W2, b2, W3, b3):
    raise NotImplementedError("write your pallas kernel here")


# speedup vs baseline: 17.0098x; 17.0098x over previous
import functools

import jax
import jax.numpy as jnp
from jax import lax
from jax.experimental import pallas as pl
from jax.experimental.pallas import tpu as pltpu
from jax.experimental.pallas import tpu_sc as plsc

_NC = 2
_NS = 16
_NW = _NC * _NS


def _sc_gather(table, idx, chunk):
    n = idx.shape[0]
    d = table.shape[1]
    per_w = n // _NW
    assert per_w * _NW == n and per_w % chunk == 0
    n_chunks = per_w // chunk

    mesh = plsc.VectorSubcoreMesh(core_axis_name="c", subcore_axis_name="s")

    @functools.partial(
        pl.kernel,
        mesh=mesh,
        out_type=jax.ShapeDtypeStruct((n, d), jnp.float32),
        scratch_types=[
            pltpu.VMEM((per_w,), jnp.int32),
            pltpu.VMEM((2, chunk, d), jnp.float32),
            pltpu.SemaphoreType.DMA,
            pltpu.SemaphoreType.DMA,
        ],
        compiler_params=pltpu.CompilerParams(use_tc_tiling_on_sc=False),
    )
    def gather_kernel(table_hbm, idx_hbm, out_hbm, idx_v, rows_v, sem0, sem1):
        wid = lax.axis_index("s") * _NC + lax.axis_index("c")
        base = wid * per_w
        pltpu.sync_copy(idx_hbm.at[pl.ds(base, per_w)], idx_v)

        sems = (sem0, sem1)

        def start(c, slot):
            pltpu.async_copy(
                table_hbm.at[idx_v.at[pl.ds(c * chunk, chunk)]],
                rows_v.at[slot],
                sems[slot],
            )

        def drain(c, slot):
            pltpu.make_async_copy(
                table_hbm.at[idx_v.at[pl.ds(c * chunk, chunk)]],
                rows_v.at[slot],
                sems[slot],
            ).wait()
            pltpu.sync_copy(
                rows_v.at[slot], out_hbm.at[pl.ds(base + c * chunk, chunk)]
            )

        start(0, 0)

        def body(cc, carry):
            c0 = 2 * cc
            start(c0 + 1, 1)
            drain(c0, 0)

            @pl.when(c0 + 2 < n_chunks)
            def _():
                start(c0 + 2, 0)

            drain(c0 + 1, 1)
            return carry

        lax.fori_loop(0, n_chunks // 2, body, 0)

        if n_chunks % 2 == 1:
            drain(n_chunks - 1, 0)

    return gather_kernel(table, idx)


def _tc_mlp(h, w1, b1, w2, b2, w3, block_m):
    batch, in_dim = h.shape
    h1 = w1.shape[1]
    h2 = w2.shape[1]
    out = w3.shape[1]

    def body(h_ref, w1_ref, b1_ref, w2_ref, b2_ref, w3_ref, o_ref):
        x = jnp.dot(h_ref[...], w1_ref[...], preferred_element_type=jnp.float32)
        x = jnp.maximum(x + b1_ref[...], 0.0)
        x = jnp.dot(x, w2_ref[...], preferred_element_type=jnp.float32)
        x = jnp.maximum(x + b2_ref[...], 0.0)
        o_ref[...] = jnp.dot(x, w3_ref[...], preferred_element_type=jnp.float32)

    return pl.pallas_call(
        body,
        grid=(batch // block_m,),
        in_specs=[
            pl.BlockSpec((block_m, in_dim), lambda i: (i, 0)),
            pl.BlockSpec((in_dim, h1), lambda i: (0, 0)),
            pl.BlockSpec((1, h1), lambda i: (0, 0)),
            pl.BlockSpec((h1, h2), lambda i: (0, 0)),
            pl.BlockSpec((1, h2), lambda i: (0, 0)),
            pl.BlockSpec((h2, out), lambda i: (0, 0)),
        ],
        out_specs=pl.BlockSpec((block_m, out), lambda i: (i, 0)),
        out_shape=jax.ShapeDtypeStruct((batch, out), jnp.float32),
    )(h, w1, b1, w2, b2, w3)


def kernel(X, table, W1, b1, W2, b2, W3, b3):
    batch, n_fields = X.shape
    d = table.shape[1]
    idx = X.reshape(-1).astype(jnp.int32)
    emb = _sc_gather(table, idx, chunk=1024)
    h = emb.reshape(batch, n_fields * d)
    y = _tc_mlp(h, W1, b1.reshape(1, -1), W2, b2.reshape(1, -1), W3, block_m=1024)
    return y + b3[None, :]
